# Initial kernel scaffold; baseline (speedup 1.0000x reference)
import jax
import jax.numpy as jnp
from jax.experimental import pallas as pl


def kernel(x, edge_index, epoch, prot, W0, b0, W1, b1, Wl1, bl1, Wl2, bl2):
    raise NotImplementedError("write your pallas kernel here")


# speedup vs baseline: 18.7476x; 18.7476x over previous
import functools

import jax
import jax.numpy as jnp
from jax import lax
from jax.experimental import pallas as pl
from jax.experimental.pallas import tpu as pltpu
from jax.experimental.pallas import tpu_sc as plsc

N = 10000
D = 128
E = 320000
C = 32
NPAD = 10240
NC = 2
NS = 16
NW = NC * NS
EPT = E // NW
K = 80
NB = EPT // K
RPT = NPAD // NS


@functools.cache
def _sc_kernels():
    mesh = plsc.VectorSubcoreMesh(core_axis_name="c", subcore_axis_name="s",
                                  num_cores=NC, num_subcores=NS)

    sc_params = pltpu.CompilerParams(needs_layout_passes=False)

    deg_kernel = functools.partial(
        pl.kernel,
        out_type=jax.ShapeDtypeStruct((NC, NPAD), jnp.float32),
        mesh=mesh,
        compiler_params=sc_params,
        scratch_types=[
            pltpu.VMEM((NB, K), jnp.int32),
            pltpu.VMEM((NPAD,), jnp.float32),
            pltpu.VMEM((NS, RPT), jnp.float32),
            pltpu.VMEM((RPT,), jnp.float32),
            pltpu.VMEM_SHARED((NS, NPAD), jnp.float32),
        ],
    )(_deg_body)

    scatter_kernel = functools.partial(
        pl.kernel,
        out_type=(jax.ShapeDtypeStruct((NPAD, D), jnp.float32),
                  jax.ShapeDtypeStruct((NPAD, D), jnp.float32)),
        mesh=mesh,
        compiler_params=sc_params,
        scratch_types=[
            pltpu.VMEM((NB, K), jnp.int32),
            pltpu.VMEM((NB, K), jnp.int32),
            pltpu.VMEM((K, D), jnp.float32),
            pltpu.VMEM_SHARED((NPAD, D), jnp.float32),
            pltpu.SemaphoreType.DMA,
        ],
    )(_scatter_body)

    return deg_kernel, scatter_kernel


def _deg_body(dst_hbm, deg_hbm, dst_v, cnt_v, red_v, sum_v, part_sh):
    cid = lax.axis_index("c")
    sid = lax.axis_index("s")
    wid = cid * NS + sid
    pltpu.sync_copy(dst_hbm.at[wid], dst_v)
    zero16 = jnp.zeros((16,), jnp.float32)

    def zb(i, carry):
        cnt_v[pl.ds(i * 16, 16)] = zero16
        return carry

    lax.fori_loop(0, NPAD // 16, zb, 0)
    one16 = jnp.ones((16,), jnp.float32)

    def cb(j, carry):
        for t in range(K // 16):
            idx = dst_v[j, pl.ds(t * 16, 16)]
            plsc.addupdate_scatter(cnt_v, [idx], one16)
        return carry

    lax.fori_loop(0, NB, cb, 0)
    pltpu.sync_copy(cnt_v, part_sh.at[sid])
    plsc.subcore_barrier()
    pltpu.sync_copy(part_sh.at[:, pl.ds(sid * RPT, RPT)], red_v)

    def rb(c, carry):
        s = red_v[0, pl.ds(c * 16, 16)]
        for j in range(1, NS):
            s = s + red_v[j, pl.ds(c * 16, 16)]
        sum_v[pl.ds(c * 16, 16)] = s
        return carry

    lax.fori_loop(0, RPT // 16, rb, 0)
    pltpu.sync_copy(sum_v, deg_hbm.at[cid, pl.ds(sid * RPT, RPT)])


def _scatter_body(h_hbm, src_hbm, dst_hbm, zero_hbm, s0_hbm, s1_hbm,
                  src_v, dst_v, rows_v, acc_sh, sem):
    cid = lax.axis_index("c")
    sid = lax.axis_index("s")
    wid = cid * NS + sid
    r0 = sid * RPT
    pltpu.sync_copy(zero_hbm.at[pl.ds(r0, RPT)], acc_sh.at[pl.ds(r0, RPT)])
    pltpu.sync_copy(src_hbm.at[wid], src_v)
    pltpu.sync_copy(dst_hbm.at[wid], dst_v)
    plsc.subcore_barrier()

    def body(j, carry):
        pltpu.async_copy(h_hbm.at[src_v.at[j]], rows_v, sem).wait()
        pltpu.sync_copy(rows_v, acc_sh.at[dst_v.at[j]], add=True)
        return carry

    lax.fori_loop(0, NB, body, 0)
    plsc.subcore_barrier()

    @pl.when(cid == 0)
    def _():
        pltpu.sync_copy(acc_sh.at[pl.ds(r0, RPT)], s0_hbm.at[pl.ds(r0, RPT)])

    @pl.when(cid == 1)
    def _():
        pltpu.sync_copy(acc_sh.at[pl.ds(r0, RPT)], s1_hbm.at[pl.ds(r0, RPT)])


BR = 1000
G = N // BR


def _l1_body(x_ref, w_ref, d0_ref, d1_ref, h_ref, dinv_ref):
    dinv = lax.rsqrt(d0_ref[...] + d1_ref[...] + 1.0)
    h = jnp.dot(x_ref[...], w_ref[...], preferred_element_type=jnp.float32)
    h_ref[...] = h * dinv
    dinv_ref[...] = dinv


def _mid_body(s0_ref, s1_ref, hp_ref, dinv_ref, b_ref, w_ref, out_ref):
    dinv = dinv_ref[...]
    u = dinv * (s0_ref[...] + s1_ref[...] + hp_ref[...]) + b_ref[...]
    r = jnp.maximum(u, 0.0)
    out_ref[...] = jnp.dot(r, w_ref[...],
                           preferred_element_type=jnp.float32) * dinv


def _norm_body(s0_ref, s1_ref, hp_ref, dinv_ref, b_ref, hn_ref):
    h = dinv_ref[...] * (s0_ref[...] + s1_ref[...] + hp_ref[...]) + b_ref[...]
    ss = jnp.sum(h * h, axis=1, keepdims=True)
    hn_ref[...] = h / jnp.sqrt(ss)


def _proto_body(hn_ref, prot_ref, wl1_ref, bl1_ref, wl2_ref, bl2_ref,
                ascl_ref, op_ref, anch_ref):
    def gb(j, carry):
        anch_ref[pl.ds(j, 1), :] = hn_ref[pl.ds(prot_ref[j], 1), :]
        return carry

    lax.fori_loop(0, 64, gb, 0)
    a = anch_ref[...]
    an = jnp.sqrt(jnp.sum(a * a, axis=1, keepdims=True))
    ascl_ref[...] = a / jnp.maximum(an, 1e-6)
    hid = jnp.maximum(
        jnp.dot(a, wl1_ref[...], preferred_element_type=jnp.float32)
        + bl1_ref[...], 0.0)
    lg = jnp.dot(hid, wl2_ref[...],
                 preferred_element_type=jnp.float32) + bl2_ref[...]
    s = lg - jnp.max(lg, axis=1, keepdims=True)
    op_ref[...] = s - jnp.log(jnp.sum(jnp.exp(s), axis=1, keepdims=True))


def _head_body(hn_ref, ascl_ref, op_ref, out_ref, xrel_ref):
    hn = hn_ref[...]
    xn = jnp.sqrt(jnp.sum(hn * hn, axis=1, keepdims=True))
    hs = hn / jnp.maximum(xn, 1e-6)
    cos = lax.dot_general(hs, ascl_ref[...], (((1,), (1,)), ((), ())),
                          preferred_element_type=jnp.float32)
    xr = (cos + 1.0) * 0.5
    xrel_ref[...] = xr
    lg = jnp.dot(xr, op_ref[...], preferred_element_type=jnp.float32)
    s = lg - jnp.max(lg, axis=1, keepdims=True)
    out_ref[...] = s - jnp.log(jnp.sum(jnp.exp(s), axis=1, keepdims=True))


def _row_spec(cols):
    return pl.BlockSpec((BR, cols), lambda i: (i, 0))


def _full_spec(rows, cols):
    return pl.BlockSpec((rows, cols), lambda i: (0, 0))


def kernel(x, edge_index, epoch, prot, W0, b0, W1, b1, Wl1, bl1, Wl2, bl2):
    src3 = edge_index[0].reshape(NW, NB, K)
    dst3 = edge_index[1].reshape(NW, NB, K)
    zeros = jnp.zeros((NPAD, D), jnp.float32)

    deg_kernel, scatter_kernel = _sc_kernels()
    degp = deg_kernel(dst3)
    d0 = degp[0, :N].reshape(N, 1)
    d1 = degp[1, :N].reshape(N, 1)

    h1p, dinv = pl.pallas_call(
        _l1_body,
        grid=(G,),
        in_specs=[_row_spec(D), _full_spec(D, D), _row_spec(1), _row_spec(1)],
        out_specs=[_row_spec(D), _row_spec(1)],
        out_shape=[jax.ShapeDtypeStruct((N, D), jnp.float32),
                   jax.ShapeDtypeStruct((N, 1), jnp.float32)],
    )(x, W0, d0, d1)

    s0, s1 = scatter_kernel(h1p, src3, dst3, zeros)

    h2p = pl.pallas_call(
        _mid_body,
        grid=(G,),
        in_specs=[_row_spec(D), _row_spec(D), _row_spec(D), _row_spec(1),
                  _full_spec(1, D), _full_spec(D, D)],
        out_specs=_row_spec(D),
        out_shape=jax.ShapeDtypeStruct((N, D), jnp.float32),
    )(s0, s1, h1p, dinv, b0.reshape(1, D), W1)

    t0, t1 = scatter_kernel(h2p, src3, dst3, zeros)

    hn = pl.pallas_call(
        _norm_body,
        grid=(G,),
        in_specs=[_row_spec(D), _row_spec(D), _row_spec(D), _row_spec(1),
                  _full_spec(1, D)],
        out_specs=_row_spec(D),
        out_shape=jax.ShapeDtypeStruct((N, D), jnp.float32),
    )(t0, t1, h2p, dinv, b1.reshape(1, D))

    ascl, op = pl.pallas_call(
        _proto_body,
        in_specs=[pl.BlockSpec(memory_space=pltpu.VMEM),
                  pl.BlockSpec(memory_space=pltpu.SMEM),
                  pl.BlockSpec(memory_space=pltpu.VMEM),
                  pl.BlockSpec(memory_space=pltpu.VMEM),
                  pl.BlockSpec(memory_space=pltpu.VMEM),
                  pl.BlockSpec(memory_space=pltpu.VMEM)],
        out_shape=[jax.ShapeDtypeStruct((64, D), jnp.float32),
                   jax.ShapeDtypeStruct((64, C), jnp.float32)],
        scratch_shapes=[pltpu.VMEM((64, D), jnp.float32)],
    )(hn, prot, Wl1, bl1.reshape(1, D), Wl2, bl2.reshape(1, C))

    out, xrel = pl.pallas_call(
        _head_body,
        grid=(G,),
        in_specs=[_row_spec(D), _full_spec(64, D), _full_spec(64, C)],
        out_specs=[pl.BlockSpec((BR, C), lambda i: (i, 0)),
                   pl.BlockSpec((BR, 64), lambda i: (i, 0))],
        out_shape=[jax.ShapeDtypeStruct((N, C), jnp.float32),
                   jax.ShapeDtypeStruct((N, 64), jnp.float32)],
    )(hn, ascl, op)

    return (out, xrel, op)
